# baseline (speedup 1.0000x reference)
import functools

import jax
import jax.numpy as jnp
from jax import lax
from jax.experimental import pallas as pl
from jax.experimental.pallas import tpu as pltpu
from jax.experimental.pallas import tpu_sc as plsc

_B, _C, _H, _W = 16, 16, 320, 320
_HW = _H * _W
_R, _L = 800, 128
_K = 25600
_SP = 0.25
_ONE_BITS = 0x3F800001

_NROW = _B * _C
_NW = 32
_CPT = _HW // _NW
_RCH = 2
_NCH = _NROW // _RCH
_NBUF = 8


def _mask_body(w_ref, mask_ref, mout_ref):
    x = w_ref[...]
    prob = jax.nn.sigmoid(x)
    xbar = jnp.mean(prob)
    r = _SP / xbar
    beta = (1.0 - _SP) / (1.0 - xbar)
    le = (r <= 1.0).astype(jnp.float32)
    resc = le * (prob * r) + (1.0 - le) * (1.0 - (1.0 - prob) * beta)
    bits = lax.bitcast_convert_type(resc, jnp.int32)

    def body(_, lohi):
        lo, hi = lohi
        mid = (lo + hi) // 2
        cnt = jnp.sum((bits >= mid).astype(jnp.int32))
        ok = cnt >= _K
        return jnp.where(ok, mid, lo), jnp.where(ok, hi, mid)

    lo, _hi = lax.fori_loop(0, 31, body, (jnp.int32(0), jnp.int32(_ONE_BITS)))
    m = (bits >= lo).astype(jnp.float32)
    mask_ref[...] = m
    mout_ref[...] = jnp.broadcast_to(m[None], (_B, _R, _L))


_sc_mesh = plsc.VectorSubcoreMesh(core_axis_name="c", subcore_axis_name="s")


@functools.partial(
    pl.kernel,
    mesh=_sc_mesh,
    out_type=jax.ShapeDtypeStruct((_NROW, _HW), jnp.float32),
    scratch_types=(
        [pltpu.VMEM((_RCH, _CPT), jnp.float32)] * (2 * _NBUF)
        + [pltpu.VMEM((_CPT,), jnp.float32)]
        + [pltpu.SemaphoreType.DMA] * (2 * _NBUF)
    ),
)
def _sc_mul(ks_hbm, mask_hbm, out_hbm, *bufs):
    ibs = bufs[0:_NBUF]
    obs = bufs[_NBUF:2 * _NBUF]
    mbuf = bufs[2 * _NBUF]
    sis = bufs[2 * _NBUF + 1:3 * _NBUF + 1]
    sos = bufs[3 * _NBUF + 1:4 * _NBUF + 1]
    wid = lax.axis_index("s") * 2 + lax.axis_index("c")
    c0 = wid * _CPT
    pltpu.sync_copy(mask_hbm.at[pl.ds(c0, _CPT)], mbuf)

    def src(c):
        return ks_hbm.at[pl.ds(c * _RCH, _RCH), pl.ds(c0, _CPT)]

    def dst(c):
        return out_hbm.at[pl.ds(c * _RCH, _RCH), pl.ds(c0, _CPT)]

    def compute(ib, ob):
        def inner(l, carry2):
            s = pl.multiple_of(l * 16, 16)
            m = mbuf[pl.ds(s, 16)]
            for r in range(_RCH):
                ob[r, pl.ds(s, 16)] = ib[r, pl.ds(s, 16)] * m
            return carry2

        lax.fori_loop(0, _CPT // 16, inner, 0)

    for b in range(_NBUF):
        pltpu.async_copy(src(b), ibs[b], sis[b])

    def round_(i, carry):
        for b in range(_NBUF):
            c = i * _NBUF + b

            @pl.when(i >= 1)
            def _():
                pltpu.make_async_copy(src(0), obs[b], sos[b]).wait()

            pltpu.make_async_copy(src(0), ibs[b], sis[b]).wait()
            compute(ibs[b], obs[b])
            pltpu.async_copy(obs[b], dst(c), sos[b])

            @pl.when(c + _NBUF < _NCH)
            def _():
                pltpu.async_copy(src(c + _NBUF), ibs[b], sis[b])

        return carry

    lax.fori_loop(0, _NCH // _NBUF, round_, 0)
    for b in range(_NBUF):
        pltpu.make_async_copy(src(0), obs[b], sos[b]).wait()


def _tc_mul_body(m_ref, ks_ref, out_ref):
    out_ref[...] = ks_ref[...] * m_ref[...][None]


def kernel(kspace, weights):
    ks = kspace.reshape(_NROW, _HW)
    ks3 = kspace.reshape(_NROW, _R, _L)
    w = weights.reshape(_R, _L)
    mask2d, mout = pl.pallas_call(
        _mask_body,
        in_specs=[pl.BlockSpec((_R, _L), lambda: (0, 0))],
        out_specs=[
            pl.BlockSpec((_R, _L), lambda: (0, 0)),
            pl.BlockSpec((_B, _R, _L), lambda: (0, 0, 0)),
        ],
        out_shape=[
            jax.ShapeDtypeStruct((_R, _L), jnp.float32),
            jax.ShapeDtypeStruct((_B, _R, _L), jnp.float32),
        ],
    )(w)
    out = _sc_mul(ks, mask2d.reshape(_HW))
    return out.reshape(_B, _C, _H, _W), mout.reshape(_B, _H, _W)


# speedup vs baseline: 1.1724x; 1.1724x over previous
import functools

import jax
import jax.numpy as jnp
from jax import lax
from jax.experimental import pallas as pl
from jax.experimental.pallas import tpu as pltpu
from jax.experimental.pallas import tpu_sc as plsc

_B, _C, _H, _W = 16, 16, 320, 320
_HW = _H * _W
_R, _L = 800, 128
_K = 25600
_SP = 0.25
_ONE_BITS = 0x3F800001

_NROW = _B * _C
_NW = 32
_CPT = _HW // _NW
_RCH = 4
_NCH = _NROW // _RCH
_NBUF = 4


def _mask_body(w_ref, mask_ref, mout_ref):
    x = w_ref[...]
    prob = jax.nn.sigmoid(x)
    xbar = jnp.mean(prob)
    r = _SP / xbar
    beta = (1.0 - _SP) / (1.0 - xbar)
    le = (r <= 1.0).astype(jnp.float32)
    resc = le * (prob * r) + (1.0 - le) * (1.0 - (1.0 - prob) * beta)
    bits = lax.bitcast_convert_type(resc, jnp.int32)

    def body(_, lohi):
        lo, hi = lohi
        mid = (lo + hi) // 2
        cnt = jnp.sum((bits >= mid).astype(jnp.int32))
        ok = cnt >= _K
        return jnp.where(ok, mid, lo), jnp.where(ok, hi, mid)

    lo, _hi = lax.fori_loop(0, 31, body, (jnp.int32(0), jnp.int32(_ONE_BITS)))
    m = (bits >= lo).astype(jnp.float32)
    mask_ref[...] = m
    mout_ref[...] = jnp.broadcast_to(m[None], (_B, _R, _L))


_sc_mesh = plsc.VectorSubcoreMesh(core_axis_name="c", subcore_axis_name="s")


@functools.partial(
    pl.kernel,
    mesh=_sc_mesh,
    out_type=jax.ShapeDtypeStruct((_NROW, _HW), jnp.float32),
    scratch_types=(
        [pltpu.VMEM((_RCH, _CPT), jnp.float32)] * (2 * _NBUF)
        + [pltpu.VMEM((_CPT,), jnp.float32)]
        + [pltpu.SemaphoreType.DMA] * (2 * _NBUF)
    ),
)
def _sc_mul(ks_hbm, mask_hbm, out_hbm, *bufs):
    ibs = bufs[0:_NBUF]
    obs = bufs[_NBUF:2 * _NBUF]
    mbuf = bufs[2 * _NBUF]
    sis = bufs[2 * _NBUF + 1:3 * _NBUF + 1]
    sos = bufs[3 * _NBUF + 1:4 * _NBUF + 1]
    wid = lax.axis_index("s") * 2 + lax.axis_index("c")
    c0 = wid * _CPT
    pltpu.sync_copy(mask_hbm.at[pl.ds(c0, _CPT)], mbuf)

    def src(c):
        return ks_hbm.at[pl.ds(c * _RCH, _RCH), pl.ds(c0, _CPT)]

    def dst(c):
        return out_hbm.at[pl.ds(c * _RCH, _RCH), pl.ds(c0, _CPT)]

    def compute(ib, ob):
        def inner(l, carry2):
            s = pl.multiple_of(l * 16, 16)
            m = mbuf[pl.ds(s, 16)]
            for r in range(_RCH):
                ob[r, pl.ds(s, 16)] = ib[r, pl.ds(s, 16)] * m
            return carry2

        lax.fori_loop(0, _CPT // 16, inner, 0)

    for b in range(_NBUF):
        pltpu.async_copy(src(b), ibs[b], sis[b])

    def round_(i, carry):
        for b in range(_NBUF):
            c = i * _NBUF + b

            @pl.when(i >= 1)
            def _():
                pltpu.make_async_copy(src(0), obs[b], sos[b]).wait()

            pltpu.make_async_copy(src(0), ibs[b], sis[b]).wait()
            compute(ibs[b], obs[b])
            pltpu.async_copy(obs[b], dst(c), sos[b])

            @pl.when(c + _NBUF < _NCH)
            def _():
                pltpu.async_copy(src(c + _NBUF), ibs[b], sis[b])

        return carry

    lax.fori_loop(0, _NCH // _NBUF, round_, 0)
    for b in range(_NBUF):
        pltpu.make_async_copy(src(0), obs[b], sos[b]).wait()


def kernel(kspace, weights):
    ks = kspace.reshape(_NROW, _HW)
    w = weights.reshape(_R, _L)
    mask2d, mout = pl.pallas_call(
        _mask_body,
        in_specs=[pl.BlockSpec((_R, _L), lambda: (0, 0))],
        out_specs=[
            pl.BlockSpec((_R, _L), lambda: (0, 0)),
            pl.BlockSpec((_B, _R, _L), lambda: (0, 0, 0)),
        ],
        out_shape=[
            jax.ShapeDtypeStruct((_R, _L), jnp.float32),
            jax.ShapeDtypeStruct((_B, _R, _L), jnp.float32),
        ],
    )(w)
    out = _sc_mul(ks, mask2d.reshape(_HW))
    return out.reshape(_B, _C, _H, _W), mout.reshape(_B, _H, _W)


# speedup vs baseline: 1.1743x; 1.0017x over previous
import functools

import jax
import jax.numpy as jnp
from jax import lax
from jax.experimental import pallas as pl
from jax.experimental.pallas import tpu as pltpu
from jax.experimental.pallas import tpu_sc as plsc

_B, _C, _H, _W = 16, 16, 320, 320
_HW = _H * _W
_R, _L = 800, 128
_K = 25600
_SP = 0.25
_ONE_BITS = 0x3F800001

_NROW = _B * _C
_NW = 32
_CPT = _HW // _NW
_RCH = 4
_NCH = _NROW // _RCH
_NBUF = 4


def _mask_body(w_ref, mask_ref, mout_ref):
    x = w_ref[...]
    prob = jax.nn.sigmoid(x)
    xbar = jnp.mean(prob)
    r = _SP / xbar
    beta = (1.0 - _SP) / (1.0 - xbar)
    le = (r <= 1.0).astype(jnp.float32)
    resc = le * (prob * r) + (1.0 - le) * (1.0 - (1.0 - prob) * beta)
    bits = lax.bitcast_convert_type(resc, jnp.int32)

    def body(_, lohi):
        lo, hi = lohi
        mid = (lo + hi) // 2
        cnt = jnp.sum((bits >= mid).astype(jnp.int32))
        ok = cnt >= _K
        return jnp.where(ok, mid, lo), jnp.where(ok, hi, mid)

    lo, _hi = lax.fori_loop(0, 31, body, (jnp.int32(0), jnp.int32(_ONE_BITS)))
    m = (bits >= lo).astype(jnp.float32)
    mask_ref[...] = m
    mout_ref[...] = jnp.broadcast_to(m[None], (_B, _R, _L))


_sc_mesh = plsc.VectorSubcoreMesh(core_axis_name="c", subcore_axis_name="s")


@functools.partial(
    pl.kernel,
    mesh=_sc_mesh,
    out_type=jax.ShapeDtypeStruct((_NROW, _HW), jnp.float32),
    scratch_types=(
        [pltpu.VMEM((_RCH, _CPT), jnp.float32)] * (2 * _NBUF)
        + [pltpu.VMEM((_CPT,), jnp.float32)]
        + [pltpu.SemaphoreType.DMA] * (2 * _NBUF)
    ),
)
def _sc_mul(ks_hbm, mask_hbm, out_hbm, *bufs):
    ibs = bufs[0:_NBUF]
    obs = bufs[_NBUF:2 * _NBUF]
    mbuf = bufs[2 * _NBUF]
    sis = bufs[2 * _NBUF + 1:3 * _NBUF + 1]
    sos = bufs[3 * _NBUF + 1:4 * _NBUF + 1]
    wid = lax.axis_index("s") * 2 + lax.axis_index("c")
    c0 = wid * _CPT
    pltpu.sync_copy(mask_hbm.at[pl.ds(c0, _CPT)], mbuf)

    def src(c):
        return ks_hbm.at[pl.ds(c * _RCH, _RCH), pl.ds(c0, _CPT)]

    def dst(c):
        return out_hbm.at[pl.ds(c * _RCH, _RCH), pl.ds(c0, _CPT)]

    def compute(ib, ob):
        def inner(l, carry2):
            s = pl.multiple_of(l * 16, 16)
            m = mbuf[pl.ds(s, 16)]
            for r in range(_RCH):
                ob[r, pl.ds(s, 16)] = ib[r, pl.ds(s, 16)] * m
            return carry2

        lax.fori_loop(0, _CPT // 16, inner, 0)

    for b in range(_NBUF):
        pltpu.async_copy(src(b), ibs[b], sis[b])

    def round_(i, carry):
        for b in range(_NBUF):
            c = i * _NBUF + b

            @pl.when(i >= 1)
            def _():
                pltpu.make_async_copy(src(0), obs[b], sos[b]).wait()

            pltpu.make_async_copy(src(0), ibs[b], sis[b]).wait()
            compute(ibs[b], obs[b])

            @pl.when(c + _NBUF < _NCH)
            def _():
                pltpu.async_copy(src(c + _NBUF), ibs[b], sis[b])

            pltpu.async_copy(obs[b], dst(c), sos[b])

        return carry

    lax.fori_loop(0, _NCH // _NBUF, round_, 0)
    for b in range(_NBUF):
        pltpu.make_async_copy(src(0), obs[b], sos[b]).wait()


def kernel(kspace, weights):
    ks = kspace.reshape(_NROW, _HW)
    w = weights.reshape(_R, _L)
    mask2d, mout = pl.pallas_call(
        _mask_body,
        in_specs=[pl.BlockSpec((_R, _L), lambda: (0, 0))],
        out_specs=[
            pl.BlockSpec((_R, _L), lambda: (0, 0)),
            pl.BlockSpec((_B, _R, _L), lambda: (0, 0, 0)),
        ],
        out_shape=[
            jax.ShapeDtypeStruct((_R, _L), jnp.float32),
            jax.ShapeDtypeStruct((_B, _R, _L), jnp.float32),
        ],
    )(w)
    out = _sc_mul(ks, mask2d.reshape(_HW))
    return out.reshape(_B, _C, _H, _W), mout.reshape(_B, _H, _W)
